# baseline (speedup 1.0000x reference)
import functools

import jax
import jax.numpy as jnp
from jax.experimental import pallas as pl
from jax.experimental.pallas import tpu as pltpu

_VMEM_LIMIT = 48 * 1024 * 1024


def _round_up(a, b):
    return (a + b - 1) // b * b


def _k1_conv1_ds(x_ref, xs_ref, w1_ref, b1_ref, wd_ref, bd_ref,
                 y1_ref, st1_ref, yd_ref, std_ref, *, wo_pad, wo):
    xb = x_ref[0].astype(jnp.bfloat16)
    y1 = jax.lax.dot_general(xb, w1_ref[...], (((0,), (0,)), ((), ())),
                             preferred_element_type=jnp.float32)
    y1 = y1 + b1_ref[...]
    y1_ref[0] = y1.astype(jnp.bfloat16)
    st1_ref[0, 0:1, :] = jnp.sum(y1, axis=0, keepdims=True)
    st1_ref[0, 1:2, :] = jnp.sum(y1 * y1, axis=0, keepdims=True)

    yd = jax.lax.dot_general(xs_ref[0], wd_ref[...], (((0,), (0,)), ((), ())),
                             preferred_element_type=jnp.float32)
    yd = yd + bd_ref[...]
    yd_ref[0] = yd.astype(jnp.bfloat16)
    row = jax.lax.broadcasted_iota(jnp.int32, (yd.shape[0], 1), 0)
    ydv = jnp.where(row % wo_pad < wo, yd, 0.0)
    std_ref[0, 0:1, :] = jnp.sum(ydv, axis=0, keepdims=True)
    std_ref[0, 1:2, :] = jnp.sum(ydv * ydv, axis=0, keepdims=True)


def _k2_conv3x3(ph_ref, w2_ref, b2_ref, y2_ref, st2_ref, *,
                ho, wo, wo_pad, hps):
    ph = ph_ref[0]
    c = ph.shape[-1]
    wins = []
    for di in range(3):
        for dj in range(3):
            v = (di % 2) * 3 + dj
            r0 = v * hps + di // 2
            wins.append(ph[r0:r0 + ho].reshape(ho * wo_pad, c))
    xw = jnp.concatenate(wins, axis=1)
    y2 = jnp.dot(xw, w2_ref[...],
                 preferred_element_type=jnp.float32) + b2_ref[...]
    y2_ref[0] = y2.astype(jnp.bfloat16)
    row = jax.lax.broadcasted_iota(jnp.int32, (y2.shape[0], 1), 0)
    y2v = jnp.where(row % wo_pad < wo, y2, 0.0)
    st2_ref[0, 0:1, :] = jnp.sum(y2v, axis=0, keepdims=True)
    st2_ref[0, 1:2, :] = jnp.sum(y2v * y2v, axis=0, keepdims=True)


def _k3_conv3(y2_ref, s2_ref, h2_ref, w3_ref, b3_ref, y3_ref, st3_ref, *,
              wo_pad, wo):
    t = y2_ref[...].astype(jnp.float32) * s2_ref[...] + h2_ref[...]
    a2 = jnp.where(t >= 0, t, 0.02 * t).astype(jnp.bfloat16)
    y3 = jnp.dot(a2, w3_ref[...],
                 preferred_element_type=jnp.float32) + b3_ref[...]
    y3_ref[...] = y3.astype(jnp.bfloat16)
    row = jax.lax.broadcasted_iota(jnp.int32, (y3.shape[0], 1), 0)
    y3v = jnp.where(row % wo_pad < wo, y3, 0.0)
    st3_ref[0, 0:1, :] = jnp.sum(y3v, axis=0, keepdims=True)
    st3_ref[0, 1:2, :] = jnp.sum(y3v * y3v, axis=0, keepdims=True)


def _k4_residual(y3_ref, s3_ref, h3_ref, yd_ref, sd_ref, hd_ref, o_ref):
    z = y3_ref[...].astype(jnp.float32) * s3_ref[...] + h3_ref[...]
    sk = yd_ref[...].astype(jnp.float32) * sd_ref[...] + hd_ref[...]
    y = z + sk
    o_ref[...] = jnp.where(y >= 0, y, 0.01 * y)


def _bn_scale_shift(gamma, beta, st, m, eps=1e-5):
    mean = st[0] / m
    var = jnp.maximum(st[1] / m - mean * mean, 0.0)
    scale = gamma / jnp.sqrt(var + eps)
    shift = beta - mean * scale
    return scale, shift


def kernel(x, w1, b1, g1, be1, w2, b2, g2, be2, w3, b3, g3, be3,
           wd, bd, gd, bed):
    n, cin, h, w = x.shape
    cout = w1.shape[1]
    c4 = w3.shape[1]
    stride = 2
    ho = (h + 2 - 3) // stride + 1
    wo = (w + 2 - 3) // stride + 1
    wo_pad = _round_up(wo, 8)
    mo = ho * wo_pad
    hw = h * w
    bf = jnp.bfloat16

    xr = x.reshape(n, cin, hw)
    xs = x[:, :, ::stride, ::stride].astype(bf)
    xs = jnp.pad(xs, ((0, 0), (0, 0), (0, 0), (0, wo_pad - wo)))
    xs = xs.reshape(n, cin, mo)

    y1, st1, yd, std = pl.pallas_call(
        functools.partial(_k1_conv1_ds, wo_pad=wo_pad, wo=wo),
        out_shape=(jax.ShapeDtypeStruct((n, hw, cout), bf),
                   jax.ShapeDtypeStruct((n, 2, cout), jnp.float32),
                   jax.ShapeDtypeStruct((n, mo, c4), bf),
                   jax.ShapeDtypeStruct((n, 2, c4), jnp.float32)),
        grid=(n,),
        in_specs=[pl.BlockSpec((1, cin, hw), lambda i: (i, 0, 0)),
                  pl.BlockSpec((1, cin, mo), lambda i: (i, 0, 0)),
                  pl.BlockSpec((cin, cout), lambda i: (0, 0)),
                  pl.BlockSpec((1, cout), lambda i: (0, 0)),
                  pl.BlockSpec((cin, c4), lambda i: (0, 0)),
                  pl.BlockSpec((1, c4), lambda i: (0, 0))],
        out_specs=(pl.BlockSpec((1, hw, cout), lambda i: (i, 0, 0)),
                   pl.BlockSpec((1, 2, cout), lambda i: (i, 0, 0)),
                   pl.BlockSpec((1, mo, c4), lambda i: (i, 0, 0)),
                   pl.BlockSpec((1, 2, c4), lambda i: (i, 0, 0))),
        compiler_params=pltpu.CompilerParams(
            dimension_semantics=("parallel",),
            vmem_limit_bytes=_VMEM_LIMIT),
    )(xr, xs, w1.astype(bf), b1.reshape(1, cout), wd.astype(bf),
      bd.reshape(1, c4))

    m1 = n * hw
    s1, h1 = _bn_scale_shift(g1, be1, jnp.sum(st1, axis=0), m1)

    a1 = y1.astype(jnp.float32) * s1 + h1
    a1 = jnp.where(a1 >= 0, a1, 0.02 * a1).astype(bf)
    a1 = a1.reshape(n, h, w, cout)
    w_pad = 2 * wo_pad + 2
    a1p = jnp.pad(a1, ((0, 0), (1, 1), (1, w_pad - w - 1), (0, 0)))
    hps = (h + 2) // 2
    variants = [a1p[:, pi::2, dj:dj + 2 * wo_pad:2, :]
                for pi in range(2) for dj in range(3)]
    ph = jnp.stack(variants, axis=1).reshape(n, 6 * hps, wo_pad, cout)

    y2, st2 = pl.pallas_call(
        functools.partial(_k2_conv3x3, ho=ho, wo=wo, wo_pad=wo_pad, hps=hps),
        out_shape=(jax.ShapeDtypeStruct((n, mo, cout), bf),
                   jax.ShapeDtypeStruct((n, 2, cout), jnp.float32)),
        grid=(n,),
        in_specs=[pl.BlockSpec((1, 6 * hps, wo_pad, cout),
                               lambda i: (i, 0, 0, 0)),
                  pl.BlockSpec((9 * cout, cout), lambda i: (0, 0)),
                  pl.BlockSpec((1, cout), lambda i: (0, 0))],
        out_specs=(pl.BlockSpec((1, mo, cout), lambda i: (i, 0, 0)),
                   pl.BlockSpec((1, 2, cout), lambda i: (i, 0, 0))),
        compiler_params=pltpu.CompilerParams(
            dimension_semantics=("parallel",),
            vmem_limit_bytes=_VMEM_LIMIT),
    )(ph, w2.reshape(9 * cout, cout).astype(bf), b2.reshape(1, cout))

    m2 = n * ho * wo
    s2, h2 = _bn_scale_shift(g2, be2, jnp.sum(st2, axis=0), m2)

    mtot = n * mo
    tm = 1024 if mtot % 1024 == 0 else mo
    nt = mtot // tm
    y3, st3 = pl.pallas_call(
        functools.partial(_k3_conv3, wo_pad=wo_pad, wo=wo),
        out_shape=(jax.ShapeDtypeStruct((mtot, c4), bf),
                   jax.ShapeDtypeStruct((nt, 2, c4), jnp.float32)),
        grid=(nt,),
        in_specs=[pl.BlockSpec((tm, cout), lambda i: (i, 0)),
                  pl.BlockSpec((1, cout), lambda i: (0, 0)),
                  pl.BlockSpec((1, cout), lambda i: (0, 0)),
                  pl.BlockSpec((cout, c4), lambda i: (0, 0)),
                  pl.BlockSpec((1, c4), lambda i: (0, 0))],
        out_specs=(pl.BlockSpec((tm, c4), lambda i: (i, 0)),
                   pl.BlockSpec((1, 2, c4), lambda i: (i, 0, 0))),
        compiler_params=pltpu.CompilerParams(
            dimension_semantics=("parallel",),
            vmem_limit_bytes=_VMEM_LIMIT),
    )(y2.reshape(mtot, cout), s2.reshape(1, cout), h2.reshape(1, cout),
      w3.astype(bf), b3.reshape(1, c4))

    s3, h3 = _bn_scale_shift(g3, be3, jnp.sum(st3, axis=0), m2)
    sd, hd = _bn_scale_shift(gd, bed, jnp.sum(std, axis=0), m2)

    out = pl.pallas_call(
        _k4_residual,
        out_shape=jax.ShapeDtypeStruct((mtot, c4), jnp.float32),
        grid=(nt,),
        in_specs=[pl.BlockSpec((tm, c4), lambda i: (i, 0)),
                  pl.BlockSpec((1, c4), lambda i: (0, 0)),
                  pl.BlockSpec((1, c4), lambda i: (0, 0)),
                  pl.BlockSpec((tm, c4), lambda i: (i, 0)),
                  pl.BlockSpec((1, c4), lambda i: (0, 0)),
                  pl.BlockSpec((1, c4), lambda i: (0, 0))],
        out_specs=pl.BlockSpec((tm, c4), lambda i: (i, 0)),
        compiler_params=pltpu.CompilerParams(
            dimension_semantics=("parallel",),
            vmem_limit_bytes=_VMEM_LIMIT),
    )(y3, s3.reshape(1, c4), h3.reshape(1, c4), yd.reshape(mtot, c4),
      sd.reshape(1, c4), hd.reshape(1, c4))

    out = out.reshape(n, ho, wo_pad, c4)[:, :, :wo, :]
    return jnp.transpose(out, (0, 3, 1, 2))
